# baseline (speedup 1.0000x reference)
import jax
import jax.numpy as jnp
from jax import lax
from jax.experimental import pallas as pl
from jax.experimental.pallas import tpu as pltpu
from jax.experimental.pallas import tpu_sc as plsc

N = 10000
R = 4
D = 128
L = 2
E = 160000
C = 2 * R
CHUNK = 128
NUM_CHUNKS = E // CHUNK
NUM_WORKERS = 32
ROWS_PER_TILE = 624
ROWS_TAIL = N - 16 * ROWS_PER_TILE


BN = 1000


def _coef(c):
    expn = 8 - 2 * (c % R) - (c // R)
    return lax.shift_left(1, expn).astype(jnp.float32)


def _transform_body(x_ref, w_ref, b_ref, y_ref):
    c = pl.program_id(1)
    y = jnp.dot(x_ref[...], w_ref[0], preferred_element_type=jnp.float32)
    y_ref[0] = (y + b_ref[0, 0]) * _coef(c)


def _transform_relu_body(ha_ref, hb_ref, w_ref, b_ref, y_ref):
    c = pl.program_id(1)
    x = jnp.maximum(ha_ref[...] + hb_ref[...], 0.0)
    y = jnp.dot(x, w_ref[0], preferred_element_type=jnp.float32)
    y_ref[0] = (y + b_ref[0, 0]) * _coef(c)


_x_spec = pl.BlockSpec((BN, D), lambda i, c: (i, 0))
_w_spec = pl.BlockSpec((1, D, D), lambda i, c: (c, 0, 0))
_b_spec = pl.BlockSpec((1, 1, D), lambda i, c: (c, 0, 0))
_y_spec = pl.BlockSpec((1, BN, D), lambda i, c: (c, i, 0))
_y_shape = jax.ShapeDtypeStruct((C, N, D), jnp.float32)

_transform = pl.pallas_call(
    _transform_body,
    grid=(N // BN, C),
    in_specs=[_x_spec, _w_spec, _b_spec],
    out_specs=_y_spec,
    out_shape=_y_shape,
)

_transform_relu = pl.pallas_call(
    _transform_relu_body,
    grid=(N // BN, C),
    in_specs=[_x_spec, _x_spec, _w_spec, _b_spec],
    out_specs=_y_spec,
    out_shape=_y_shape,
)


def _add_body(a_ref, b_ref, o_ref):
    o_ref[...] = a_ref[...] + b_ref[...]

_add = pl.pallas_call(
    _add_body,
    grid=(N // BN,),
    in_specs=[pl.BlockSpec((BN, D), lambda i: (i, 0))] * 2,
    out_specs=pl.BlockSpec((BN, D), lambda i: (i, 0)),
    out_shape=jax.ShapeDtypeStruct((N, D), jnp.float32),
)


NK_LO = NUM_CHUNKS // NUM_WORKERS
NUM_HI = NUM_CHUNKS - NK_LO * NUM_WORKERS
E_LO = NK_LO * CHUNK


def _sc_body(y_hbm, ei_hbm, et_hbm, zeros_hbm, out0_hbm, out1_hbm,
             e0_s, e1_s, et_s,
             gi0, gi1, di0, di1, r0, r1, acc, gsem, ssem):
    core = lax.axis_index("c")
    sub = lax.axis_index("s")
    wid = sub * 2 + core

    base = sub * ROWS_PER_TILE
    pltpu.sync_copy(zeros_hbm.at[pl.ds(base, ROWS_PER_TILE)],
                    acc.at[pl.ds(base, ROWS_PER_TILE)])

    @pl.when(sub == 15)
    def _():
        pltpu.sync_copy(zeros_hbm.at[pl.ds(16 * ROWS_PER_TILE, ROWS_TAIL)],
                        acc.at[pl.ds(16 * ROWS_PER_TILE, ROWS_TAIL)])

    is_hi = wid < NUM_HI
    nk = jnp.where(is_hi, NK_LO + 1, NK_LO)
    start = jnp.where(is_hi, wid * (NK_LO + 1),
                      NUM_HI * (NK_LO + 1) + (wid - NUM_HI) * NK_LO)
    ebase = pl.multiple_of(start * CHUNK, CHUNK)
    pltpu.sync_copy(ei_hbm.at[0, pl.ds(ebase, E_LO)], e0_s.at[pl.ds(0, E_LO)])
    pltpu.sync_copy(ei_hbm.at[1, pl.ds(ebase, E_LO)], e1_s.at[pl.ds(0, E_LO)])
    pltpu.sync_copy(et_hbm.at[pl.ds(ebase, E_LO)], et_s.at[pl.ds(0, E_LO)])

    @pl.when(is_hi)
    def _():
        off = pl.multiple_of(ebase + E_LO, CHUNK)
        pltpu.sync_copy(ei_hbm.at[0, pl.ds(off, CHUNK)],
                        e0_s.at[pl.ds(E_LO, CHUNK)])
        pltpu.sync_copy(ei_hbm.at[1, pl.ds(off, CHUNK)],
                        e1_s.at[pl.ds(E_LO, CHUNK)])
        pltpu.sync_copy(et_hbm.at[pl.ds(off, CHUNK)],
                        et_s.at[pl.ds(E_LO, CHUNK)])

    plsc.subcore_barrier()

    gis = [gi0, gi1]
    dis = [di0, di1]
    rows = [r0, r1]

    def build(k, d):
        koff = k * CHUNK
        for j in range(CHUNK // 16):
            sl = pl.ds(16 * j, 16)
            ksl = pl.ds(koff + 16 * j, 16)
            t = et_s[ksl]
            a = e0_s[ksl]
            bb = e1_s[ksl]
            if d == 0:
                gis[0][sl] = t * N + bb
                dis[0][sl] = a
            else:
                gis[1][sl] = (t + R) * N + a
                dis[1][sl] = bb

    def gissue(d):
        pltpu.async_copy(y_hbm.at[gis[d]], rows[d], gsem)

    def gwait(d):
        pltpu.make_async_copy(y_hbm.at[gis[d]], rows[d], gsem).wait()

    def sissue(d):
        pltpu.async_copy(rows[d], acc.at[dis[d]], ssem, add=True)

    def swait(d):
        pltpu.make_async_copy(rows[d], acc.at[dis[d]], ssem).wait()

    nu = 2 * nk
    build(0, 0)
    gissue(0)

    def pair_body(m, carry):
        for b in (0, 1):
            u = 2 * m + b
            un = u + 1
            nb = 1 - b

            @pl.when(jnp.logical_and(un < nu, un >= 2))
            def _():
                swait(nb)

            @pl.when(un < nu)
            def _():
                build(m + b, nb)
                gissue(nb)

            @pl.when(u < nu)
            def _():
                gwait(b)
                sissue(b)

        return carry

    lax.fori_loop(0, NK_LO + 1, pair_body, 0)
    swait(0)
    swait(1)
    plsc.subcore_barrier()

    @pl.when(core == 0)
    def _():
        pltpu.sync_copy(acc.at[pl.ds(base, ROWS_PER_TILE)],
                        out0_hbm.at[pl.ds(base, ROWS_PER_TILE)])

        @pl.when(sub == 15)
        def _():
            pltpu.sync_copy(acc.at[pl.ds(16 * ROWS_PER_TILE, ROWS_TAIL)],
                            out0_hbm.at[pl.ds(16 * ROWS_PER_TILE, ROWS_TAIL)])

    @pl.when(core == 1)
    def _():
        pltpu.sync_copy(acc.at[pl.ds(base, ROWS_PER_TILE)],
                        out1_hbm.at[pl.ds(base, ROWS_PER_TILE)])

        @pl.when(sub == 15)
        def _():
            pltpu.sync_copy(acc.at[pl.ds(16 * ROWS_PER_TILE, ROWS_TAIL)],
                            out1_hbm.at[pl.ds(16 * ROWS_PER_TILE, ROWS_TAIL)])


_sc_scatter = pl.kernel(
    _sc_body,
    out_type=(jax.ShapeDtypeStruct((N, D), jnp.float32),
              jax.ShapeDtypeStruct((N, D), jnp.float32)),
    mesh=plsc.VectorSubcoreMesh(core_axis_name="c", subcore_axis_name="s"),
    scratch_types=[
        pltpu.VMEM((E_LO + CHUNK,), jnp.int32),
        pltpu.VMEM((E_LO + CHUNK,), jnp.int32),
        pltpu.VMEM((E_LO + CHUNK,), jnp.int32),
        pltpu.VMEM((CHUNK,), jnp.int32),
        pltpu.VMEM((CHUNK,), jnp.int32),
        pltpu.VMEM((CHUNK,), jnp.int32),
        pltpu.VMEM((CHUNK,), jnp.int32),
        pltpu.VMEM((CHUNK, D), jnp.float32),
        pltpu.VMEM((CHUNK, D), jnp.float32),
        pltpu.VMEM_SHARED((N, D), jnp.float32),
        pltpu.SemaphoreType.DMA,
        pltpu.SemaphoreType.DMA,
    ],
)


@jax.jit
def kernel(edge_index, edge_type, embeddings, W, b):
    zeros = jnp.zeros((N, D), jnp.float32)
    b3 = b.reshape(L, C, 1, D)

    y = _transform(embeddings, W[0], b3[0])
    ha, hb = _sc_scatter(y.reshape(C * N, D), edge_index, edge_type, zeros)

    y = _transform_relu(ha, hb, W[1], b3[1])
    ha, hb = _sc_scatter(y.reshape(C * N, D), edge_index, edge_type, zeros)

    return _add(ha, hb)


# speedup vs baseline: 1.2551x; 1.2551x over previous
import jax
import jax.numpy as jnp
from jax import lax
from jax.experimental import pallas as pl
from jax.experimental.pallas import tpu as pltpu
from jax.experimental.pallas import tpu_sc as plsc

N = 10000
R = 4
D = 128
L = 2
E = 160000
C = 2 * R
CHUNK = 128
NUM_CHUNKS = E // CHUNK
NUM_WORKERS = 32
ROWS_PER_TILE = 624
ROWS_TAIL = N - 16 * ROWS_PER_TILE


BN = 5000


def _coef(c):
    expn = 8 - 2 * (c % R) - (c // R)
    return lax.shift_left(1, expn).astype(jnp.float32)


def _transform_body(x_ref, w_ref, b_ref, y_ref):
    c = pl.program_id(1)
    y = jnp.dot(x_ref[...], w_ref[0], preferred_element_type=jnp.float32)
    y_ref[0] = (y + b_ref[0, 0]) * _coef(c)


def _transform_relu_body(ha_ref, hb_ref, w_ref, b_ref, y_ref):
    c = pl.program_id(1)
    x = jnp.maximum(ha_ref[...] + hb_ref[...], 0.0)
    y = jnp.dot(x, w_ref[0], preferred_element_type=jnp.float32)
    y_ref[0] = (y + b_ref[0, 0]) * _coef(c)


_x_spec = pl.BlockSpec((BN, D), lambda i, c: (i, 0))
_w_spec = pl.BlockSpec((1, D, D), lambda i, c: (c, 0, 0))
_b_spec = pl.BlockSpec((1, 1, D), lambda i, c: (c, 0, 0))
_y_spec = pl.BlockSpec((1, BN, D), lambda i, c: (c, i, 0))
_y_shape = jax.ShapeDtypeStruct((C, N, D), jnp.float32)

_transform = pl.pallas_call(
    _transform_body,
    grid=(N // BN, C),
    in_specs=[_x_spec, _w_spec, _b_spec],
    out_specs=_y_spec,
    out_shape=_y_shape,
)

_transform_relu = pl.pallas_call(
    _transform_relu_body,
    grid=(N // BN, C),
    in_specs=[_x_spec, _x_spec, _w_spec, _b_spec],
    out_specs=_y_spec,
    out_shape=_y_shape,
)


def _add_body(a_ref, b_ref, o_ref):
    o_ref[...] = a_ref[...] + b_ref[...]

_add = pl.pallas_call(
    _add_body,
    grid=(N // BN,),
    in_specs=[pl.BlockSpec((BN, D), lambda i: (i, 0))] * 2,
    out_specs=pl.BlockSpec((BN, D), lambda i: (i, 0)),
    out_shape=jax.ShapeDtypeStruct((N, D), jnp.float32),
)


NK_LO = NUM_CHUNKS // NUM_WORKERS
NUM_HI = NUM_CHUNKS - NK_LO * NUM_WORKERS
E_LO = NK_LO * CHUNK


def _sc_body(y_hbm, ei_hbm, et_hbm, zeros_hbm, out0_hbm, out1_hbm,
             e0_s, e1_s, et_s,
             gi0, gi1, di0, di1, r0, r1, acc, gsem, ssem):
    core = lax.axis_index("c")
    sub = lax.axis_index("s")
    wid = sub * 2 + core

    base = sub * ROWS_PER_TILE
    pltpu.sync_copy(zeros_hbm.at[pl.ds(base, ROWS_PER_TILE)],
                    acc.at[pl.ds(base, ROWS_PER_TILE)])

    @pl.when(sub == 15)
    def _():
        pltpu.sync_copy(zeros_hbm.at[pl.ds(16 * ROWS_PER_TILE, ROWS_TAIL)],
                        acc.at[pl.ds(16 * ROWS_PER_TILE, ROWS_TAIL)])

    is_hi = wid < NUM_HI
    nk = jnp.where(is_hi, NK_LO + 1, NK_LO)
    start = jnp.where(is_hi, wid * (NK_LO + 1),
                      NUM_HI * (NK_LO + 1) + (wid - NUM_HI) * NK_LO)
    ebase = pl.multiple_of(start * CHUNK, CHUNK)
    pltpu.sync_copy(ei_hbm.at[0, pl.ds(ebase, E_LO)], e0_s.at[pl.ds(0, E_LO)])
    pltpu.sync_copy(ei_hbm.at[1, pl.ds(ebase, E_LO)], e1_s.at[pl.ds(0, E_LO)])
    pltpu.sync_copy(et_hbm.at[pl.ds(ebase, E_LO)], et_s.at[pl.ds(0, E_LO)])

    @pl.when(is_hi)
    def _():
        off = pl.multiple_of(ebase + E_LO, CHUNK)
        pltpu.sync_copy(ei_hbm.at[0, pl.ds(off, CHUNK)],
                        e0_s.at[pl.ds(E_LO, CHUNK)])
        pltpu.sync_copy(ei_hbm.at[1, pl.ds(off, CHUNK)],
                        e1_s.at[pl.ds(E_LO, CHUNK)])
        pltpu.sync_copy(et_hbm.at[pl.ds(off, CHUNK)],
                        et_s.at[pl.ds(E_LO, CHUNK)])

    plsc.subcore_barrier()

    gis = [gi0, gi1]
    dis = [di0, di1]
    rows = [r0, r1]

    def build(k, d):
        koff = k * CHUNK
        for j in range(CHUNK // 16):
            sl = pl.ds(16 * j, 16)
            ksl = pl.ds(koff + 16 * j, 16)
            t = et_s[ksl]
            a = e0_s[ksl]
            bb = e1_s[ksl]
            if d == 0:
                gis[0][sl] = t * N + bb
                dis[0][sl] = a
            else:
                gis[1][sl] = (t + R) * N + a
                dis[1][sl] = bb

    def gissue(d):
        pltpu.async_copy(y_hbm.at[gis[d]], rows[d], gsem)

    def gwait(d):
        pltpu.make_async_copy(y_hbm.at[gis[d]], rows[d], gsem).wait()

    def sissue(d):
        pltpu.async_copy(rows[d], acc.at[dis[d]], ssem, add=True)

    def swait(d):
        pltpu.make_async_copy(rows[d], acc.at[dis[d]], ssem).wait()

    nu = 2 * nk
    build(0, 0)
    gissue(0)

    def pair_body(m, carry):
        for b in (0, 1):
            u = 2 * m + b
            un = u + 1
            nb = 1 - b

            @pl.when(jnp.logical_and(un < nu, un >= 2))
            def _():
                swait(nb)

            @pl.when(un < nu)
            def _():
                build(m + b, nb)
                gissue(nb)

            @pl.when(u < nu)
            def _():
                gwait(b)
                sissue(b)

        return carry

    lax.fori_loop(0, NK_LO + 1, pair_body, 0)
    swait(0)
    swait(1)
    plsc.subcore_barrier()

    @pl.when(core == 0)
    def _():
        pltpu.sync_copy(acc.at[pl.ds(base, ROWS_PER_TILE)],
                        out0_hbm.at[pl.ds(base, ROWS_PER_TILE)])

        @pl.when(sub == 15)
        def _():
            pltpu.sync_copy(acc.at[pl.ds(16 * ROWS_PER_TILE, ROWS_TAIL)],
                            out0_hbm.at[pl.ds(16 * ROWS_PER_TILE, ROWS_TAIL)])

    @pl.when(core == 1)
    def _():
        pltpu.sync_copy(acc.at[pl.ds(base, ROWS_PER_TILE)],
                        out1_hbm.at[pl.ds(base, ROWS_PER_TILE)])

        @pl.when(sub == 15)
        def _():
            pltpu.sync_copy(acc.at[pl.ds(16 * ROWS_PER_TILE, ROWS_TAIL)],
                            out1_hbm.at[pl.ds(16 * ROWS_PER_TILE, ROWS_TAIL)])


_sc_scatter = pl.kernel(
    _sc_body,
    out_type=(jax.ShapeDtypeStruct((N, D), jnp.float32),
              jax.ShapeDtypeStruct((N, D), jnp.float32)),
    mesh=plsc.VectorSubcoreMesh(core_axis_name="c", subcore_axis_name="s"),
    scratch_types=[
        pltpu.VMEM((E_LO + CHUNK,), jnp.int32),
        pltpu.VMEM((E_LO + CHUNK,), jnp.int32),
        pltpu.VMEM((E_LO + CHUNK,), jnp.int32),
        pltpu.VMEM((CHUNK,), jnp.int32),
        pltpu.VMEM((CHUNK,), jnp.int32),
        pltpu.VMEM((CHUNK,), jnp.int32),
        pltpu.VMEM((CHUNK,), jnp.int32),
        pltpu.VMEM((CHUNK, D), jnp.float32),
        pltpu.VMEM((CHUNK, D), jnp.float32),
        pltpu.VMEM_SHARED((N, D), jnp.float32),
        pltpu.SemaphoreType.DMA,
        pltpu.SemaphoreType.DMA,
    ],
)


@jax.jit
def kernel(edge_index, edge_type, embeddings, W, b):
    zeros = jnp.zeros((N, D), jnp.float32)
    b3 = b.reshape(L, C, 1, D)

    y = _transform(embeddings, W[0], b3[0])
    ha, hb = _sc_scatter(y.reshape(C * N, D), edge_index, edge_type, zeros)

    y = _transform_relu(ha, hb, W[1], b3[1])
    ha, hb = _sc_scatter(y.reshape(C * N, D), edge_index, edge_type, zeros)

    return _add(ha, hb)


# speedup vs baseline: 1.3158x; 1.0484x over previous
import jax
import jax.numpy as jnp
from jax import lax
from jax.experimental import pallas as pl
from jax.experimental.pallas import tpu as pltpu
from jax.experimental.pallas import tpu_sc as plsc

N = 10000
R = 4
D = 128
L = 2
E = 160000
C = 2 * R
CHUNK = 128
NUM_CHUNKS = E // CHUNK
NUM_WORKERS = 32
ROWS_PER_TILE = 624
ROWS_TAIL = N - 16 * ROWS_PER_TILE


BN = 10000


def _coef(c):
    expn = 8 - 2 * (c % R) - (c // R)
    return lax.shift_left(1, expn).astype(jnp.float32)


def _transform_body(x_ref, w_ref, b_ref, y_ref):
    c = pl.program_id(1)
    y = jnp.dot(x_ref[...], w_ref[0], preferred_element_type=jnp.float32)
    y_ref[0] = (y + b_ref[0, 0]) * _coef(c)


def _transform_relu_body(ha_ref, hb_ref, w_ref, b_ref, y_ref):
    c = pl.program_id(1)
    x = jnp.maximum(ha_ref[...] + hb_ref[...], 0.0)
    y = jnp.dot(x, w_ref[0], preferred_element_type=jnp.float32)
    y_ref[0] = (y + b_ref[0, 0]) * _coef(c)


_x_spec = pl.BlockSpec((BN, D), lambda i, c: (i, 0))
_w_spec = pl.BlockSpec((1, D, D), lambda i, c: (c, 0, 0))
_b_spec = pl.BlockSpec((1, 1, D), lambda i, c: (c, 0, 0))
_y_spec = pl.BlockSpec((1, BN, D), lambda i, c: (c, i, 0))
_y_shape = jax.ShapeDtypeStruct((C, N, D), jnp.float32)

_transform = pl.pallas_call(
    _transform_body,
    grid=(N // BN, C),
    in_specs=[_x_spec, _w_spec, _b_spec],
    out_specs=_y_spec,
    out_shape=_y_shape,
)

_transform_relu = pl.pallas_call(
    _transform_relu_body,
    grid=(N // BN, C),
    in_specs=[_x_spec, _x_spec, _w_spec, _b_spec],
    out_specs=_y_spec,
    out_shape=_y_shape,
)


def _add_body(a_ref, b_ref, o_ref):
    o_ref[...] = a_ref[...] + b_ref[...]

_add = pl.pallas_call(
    _add_body,
    grid=(N // BN,),
    in_specs=[pl.BlockSpec((BN, D), lambda i: (i, 0))] * 2,
    out_specs=pl.BlockSpec((BN, D), lambda i: (i, 0)),
    out_shape=jax.ShapeDtypeStruct((N, D), jnp.float32),
)


NK_LO = NUM_CHUNKS // NUM_WORKERS
NUM_HI = NUM_CHUNKS - NK_LO * NUM_WORKERS
E_LO = NK_LO * CHUNK


def _sc_body(y_hbm, ei_hbm, et_hbm, zeros_hbm, out0_hbm, out1_hbm,
             e0_s, e1_s, et_s,
             gi0, gi1, di0, di1, r0, r1, acc, gsem, ssem):
    core = lax.axis_index("c")
    sub = lax.axis_index("s")
    wid = sub * 2 + core

    base = sub * ROWS_PER_TILE
    pltpu.sync_copy(zeros_hbm.at[pl.ds(base, ROWS_PER_TILE)],
                    acc.at[pl.ds(base, ROWS_PER_TILE)])

    @pl.when(sub == 15)
    def _():
        pltpu.sync_copy(zeros_hbm.at[pl.ds(16 * ROWS_PER_TILE, ROWS_TAIL)],
                        acc.at[pl.ds(16 * ROWS_PER_TILE, ROWS_TAIL)])

    is_hi = wid < NUM_HI
    nk = jnp.where(is_hi, NK_LO + 1, NK_LO)
    start = jnp.where(is_hi, wid * (NK_LO + 1),
                      NUM_HI * (NK_LO + 1) + (wid - NUM_HI) * NK_LO)
    ebase = pl.multiple_of(start * CHUNK, CHUNK)
    pltpu.sync_copy(ei_hbm.at[0, pl.ds(ebase, E_LO)], e0_s.at[pl.ds(0, E_LO)])
    pltpu.sync_copy(ei_hbm.at[1, pl.ds(ebase, E_LO)], e1_s.at[pl.ds(0, E_LO)])
    pltpu.sync_copy(et_hbm.at[pl.ds(ebase, E_LO)], et_s.at[pl.ds(0, E_LO)])

    @pl.when(is_hi)
    def _():
        off = pl.multiple_of(ebase + E_LO, CHUNK)
        pltpu.sync_copy(ei_hbm.at[0, pl.ds(off, CHUNK)],
                        e0_s.at[pl.ds(E_LO, CHUNK)])
        pltpu.sync_copy(ei_hbm.at[1, pl.ds(off, CHUNK)],
                        e1_s.at[pl.ds(E_LO, CHUNK)])
        pltpu.sync_copy(et_hbm.at[pl.ds(off, CHUNK)],
                        et_s.at[pl.ds(E_LO, CHUNK)])

    plsc.subcore_barrier()

    gis = [gi0, gi1]
    dis = [di0, di1]
    rows = [r0, r1]

    def build(k, d):
        koff = k * CHUNK
        for j in range(CHUNK // 16):
            sl = pl.ds(16 * j, 16)
            ksl = pl.ds(koff + 16 * j, 16)
            t = et_s[ksl]
            a = e0_s[ksl]
            bb = e1_s[ksl]
            if d == 0:
                gis[0][sl] = t * N + bb
                dis[0][sl] = a
            else:
                gis[1][sl] = (t + R) * N + a
                dis[1][sl] = bb

    def gissue(d):
        pltpu.async_copy(y_hbm.at[gis[d]], rows[d], gsem)

    def gwait(d):
        pltpu.make_async_copy(y_hbm.at[gis[d]], rows[d], gsem).wait()

    def sissue(d):
        pltpu.async_copy(rows[d], acc.at[dis[d]], ssem, add=True)

    def swait(d):
        pltpu.make_async_copy(rows[d], acc.at[dis[d]], ssem).wait()

    nu = 2 * nk
    build(0, 0)
    gissue(0)

    def pair_body(m, carry):
        for b in (0, 1):
            u = 2 * m + b
            un = u + 1
            nb = 1 - b

            @pl.when(jnp.logical_and(un < nu, un >= 2))
            def _():
                swait(nb)

            @pl.when(un < nu)
            def _():
                build(m + b, nb)
                gissue(nb)

            @pl.when(u < nu)
            def _():
                gwait(b)
                sissue(b)

        return carry

    lax.fori_loop(0, NK_LO + 1, pair_body, 0)
    swait(0)
    swait(1)
    plsc.subcore_barrier()

    @pl.when(core == 0)
    def _():
        pltpu.sync_copy(acc.at[pl.ds(base, ROWS_PER_TILE)],
                        out0_hbm.at[pl.ds(base, ROWS_PER_TILE)])

        @pl.when(sub == 15)
        def _():
            pltpu.sync_copy(acc.at[pl.ds(16 * ROWS_PER_TILE, ROWS_TAIL)],
                            out0_hbm.at[pl.ds(16 * ROWS_PER_TILE, ROWS_TAIL)])

    @pl.when(core == 1)
    def _():
        pltpu.sync_copy(acc.at[pl.ds(base, ROWS_PER_TILE)],
                        out1_hbm.at[pl.ds(base, ROWS_PER_TILE)])

        @pl.when(sub == 15)
        def _():
            pltpu.sync_copy(acc.at[pl.ds(16 * ROWS_PER_TILE, ROWS_TAIL)],
                            out1_hbm.at[pl.ds(16 * ROWS_PER_TILE, ROWS_TAIL)])


_sc_scatter = pl.kernel(
    _sc_body,
    out_type=(jax.ShapeDtypeStruct((N, D), jnp.float32),
              jax.ShapeDtypeStruct((N, D), jnp.float32)),
    mesh=plsc.VectorSubcoreMesh(core_axis_name="c", subcore_axis_name="s"),
    scratch_types=[
        pltpu.VMEM((E_LO + CHUNK,), jnp.int32),
        pltpu.VMEM((E_LO + CHUNK,), jnp.int32),
        pltpu.VMEM((E_LO + CHUNK,), jnp.int32),
        pltpu.VMEM((CHUNK,), jnp.int32),
        pltpu.VMEM((CHUNK,), jnp.int32),
        pltpu.VMEM((CHUNK,), jnp.int32),
        pltpu.VMEM((CHUNK,), jnp.int32),
        pltpu.VMEM((CHUNK, D), jnp.float32),
        pltpu.VMEM((CHUNK, D), jnp.float32),
        pltpu.VMEM_SHARED((N, D), jnp.float32),
        pltpu.SemaphoreType.DMA,
        pltpu.SemaphoreType.DMA,
    ],
)


@jax.jit
def kernel(edge_index, edge_type, embeddings, W, b):
    zeros = jnp.zeros((N, D), jnp.float32)
    b3 = b.reshape(L, C, 1, D)

    y = _transform(embeddings, W[0], b3[0])
    ha, hb = _sc_scatter(y.reshape(C * N, D), edge_index, edge_type, zeros)

    y = _transform_relu(ha, hb, W[1], b3[1])
    ha, hb = _sc_scatter(y.reshape(C * N, D), edge_index, edge_type, zeros)

    return _add(ha, hb)


# speedup vs baseline: 1.3359x; 1.0153x over previous
import jax
import jax.numpy as jnp
from jax import lax
from jax.experimental import pallas as pl
from jax.experimental.pallas import tpu as pltpu
from jax.experimental.pallas import tpu_sc as plsc

N = 10000
R = 4
D = 128
L = 2
E = 160000
C = 2 * R
CHUNK = 128
NUM_CHUNKS = E // CHUNK
NUM_WORKERS = 32
ROWS_PER_TILE = 624
ROWS_TAIL = N - 16 * ROWS_PER_TILE


BN = 10000


def _coef(c):
    expn = 8 - 2 * (c % R) - (c // R)
    return lax.shift_left(1, expn).astype(jnp.float32)


_x_spec = pl.BlockSpec((BN, D), lambda i, c: (i, 0))
_y_spec = pl.BlockSpec((1, BN, D), lambda i, c: (c, i, 0))
_y_shape = jax.ShapeDtypeStruct((C, N, D), jnp.float32)


def _w_spec(l):
    return pl.BlockSpec((1, 1, D, D), lambda i, c: (l, c, 0, 0))


def _b_spec(l):
    return pl.BlockSpec((1, 1, 1, D), lambda i, c: (l, c, 0, 0))


def _transform_body(x_ref, w_ref, b_ref, y_ref):
    c = pl.program_id(1)
    y = jnp.dot(x_ref[...], w_ref[0, 0], preferred_element_type=jnp.float32)
    y_ref[0] = (y + b_ref[0, 0, 0]) * _coef(c)


def _transform_relu_body(ha_ref, hb_ref, w_ref, b_ref, y_ref):
    c = pl.program_id(1)
    x = jnp.maximum(ha_ref[...] + hb_ref[...], 0.0)
    y = jnp.dot(x, w_ref[0, 0], preferred_element_type=jnp.float32)
    y_ref[0] = (y + b_ref[0, 0, 0]) * _coef(c)


_transform = pl.pallas_call(
    _transform_body,
    grid=(N // BN, C),
    in_specs=[_x_spec, _w_spec(0), _b_spec(0)],
    out_specs=_y_spec,
    out_shape=_y_shape,
)

_transform_relu = pl.pallas_call(
    _transform_relu_body,
    grid=(N // BN, C),
    in_specs=[_x_spec, _x_spec, _w_spec(1), _b_spec(1)],
    out_specs=_y_spec,
    out_shape=_y_shape,
)


def _add_body(a_ref, b_ref, o_ref):
    o_ref[...] = a_ref[...] + b_ref[...]

_add = pl.pallas_call(
    _add_body,
    grid=(N // BN,),
    in_specs=[pl.BlockSpec((BN, D), lambda i: (i, 0))] * 2,
    out_specs=pl.BlockSpec((BN, D), lambda i: (i, 0)),
    out_shape=jax.ShapeDtypeStruct((N, D), jnp.float32),
)


NK_LO = NUM_CHUNKS // NUM_WORKERS
NUM_HI = NUM_CHUNKS - NK_LO * NUM_WORKERS
E_LO = NK_LO * CHUNK


def _sc_body(y_hbm, ei_hbm, et_hbm, zeros_hbm, out0_hbm, out1_hbm,
             e0_s, e1_s, et_s,
             gi0, gi1, di0, di1, r0, r1, acc, gsem, ssem):
    core = lax.axis_index("c")
    sub = lax.axis_index("s")
    wid = sub * 2 + core

    base = sub * ROWS_PER_TILE
    pltpu.sync_copy(zeros_hbm.at[pl.ds(0, ROWS_PER_TILE)],
                    acc.at[pl.ds(base, ROWS_PER_TILE)])

    @pl.when(sub == 15)
    def _():
        pltpu.sync_copy(zeros_hbm.at[pl.ds(0, ROWS_TAIL)],
                        acc.at[pl.ds(16 * ROWS_PER_TILE, ROWS_TAIL)])

    is_hi = wid < NUM_HI
    nk = jnp.where(is_hi, NK_LO + 1, NK_LO)
    start = jnp.where(is_hi, wid * (NK_LO + 1),
                      NUM_HI * (NK_LO + 1) + (wid - NUM_HI) * NK_LO)
    ebase = pl.multiple_of(start * CHUNK, CHUNK)
    pltpu.sync_copy(ei_hbm.at[0, pl.ds(ebase, E_LO)], e0_s.at[pl.ds(0, E_LO)])
    pltpu.sync_copy(ei_hbm.at[1, pl.ds(ebase, E_LO)], e1_s.at[pl.ds(0, E_LO)])
    pltpu.sync_copy(et_hbm.at[pl.ds(ebase, E_LO)], et_s.at[pl.ds(0, E_LO)])

    @pl.when(is_hi)
    def _():
        off = pl.multiple_of(ebase + E_LO, CHUNK)
        pltpu.sync_copy(ei_hbm.at[0, pl.ds(off, CHUNK)],
                        e0_s.at[pl.ds(E_LO, CHUNK)])
        pltpu.sync_copy(ei_hbm.at[1, pl.ds(off, CHUNK)],
                        e1_s.at[pl.ds(E_LO, CHUNK)])
        pltpu.sync_copy(et_hbm.at[pl.ds(off, CHUNK)],
                        et_s.at[pl.ds(E_LO, CHUNK)])

    plsc.subcore_barrier()

    gis = [gi0, gi1]
    dis = [di0, di1]
    rows = [r0, r1]

    def build(k, d):
        koff = k * CHUNK
        for j in range(CHUNK // 16):
            sl = pl.ds(16 * j, 16)
            ksl = pl.ds(koff + 16 * j, 16)
            t = et_s[ksl]
            a = e0_s[ksl]
            bb = e1_s[ksl]
            if d == 0:
                gis[0][sl] = t * N + bb
                dis[0][sl] = a
            else:
                gis[1][sl] = (t + R) * N + a
                dis[1][sl] = bb

    def gissue(d):
        pltpu.async_copy(y_hbm.at[gis[d]], rows[d], gsem)

    def gwait(d):
        pltpu.make_async_copy(y_hbm.at[gis[d]], rows[d], gsem).wait()

    def sissue(d):
        pltpu.async_copy(rows[d], acc.at[dis[d]], ssem, add=True)

    def swait(d):
        pltpu.make_async_copy(rows[d], acc.at[dis[d]], ssem).wait()

    nu = 2 * nk
    build(0, 0)
    gissue(0)

    def pair_body(m, carry):
        for b in (0, 1):
            u = 2 * m + b
            un = u + 1
            nb = 1 - b

            @pl.when(jnp.logical_and(un < nu, un >= 2))
            def _():
                swait(nb)

            @pl.when(un < nu)
            def _():
                build(m + b, nb)
                gissue(nb)

            @pl.when(u < nu)
            def _():
                gwait(b)
                sissue(b)

        return carry

    lax.fori_loop(0, NK_LO + 1, pair_body, 0)
    swait(0)
    swait(1)
    plsc.subcore_barrier()

    @pl.when(core == 0)
    def _():
        pltpu.sync_copy(acc.at[pl.ds(base, ROWS_PER_TILE)],
                        out0_hbm.at[pl.ds(base, ROWS_PER_TILE)])

        @pl.when(sub == 15)
        def _():
            pltpu.sync_copy(acc.at[pl.ds(16 * ROWS_PER_TILE, ROWS_TAIL)],
                            out0_hbm.at[pl.ds(16 * ROWS_PER_TILE, ROWS_TAIL)])

    @pl.when(core == 1)
    def _():
        pltpu.sync_copy(acc.at[pl.ds(base, ROWS_PER_TILE)],
                        out1_hbm.at[pl.ds(base, ROWS_PER_TILE)])

        @pl.when(sub == 15)
        def _():
            pltpu.sync_copy(acc.at[pl.ds(16 * ROWS_PER_TILE, ROWS_TAIL)],
                            out1_hbm.at[pl.ds(16 * ROWS_PER_TILE, ROWS_TAIL)])


_sc_scatter = pl.kernel(
    _sc_body,
    out_type=(jax.ShapeDtypeStruct((N, D), jnp.float32),
              jax.ShapeDtypeStruct((N, D), jnp.float32)),
    mesh=plsc.VectorSubcoreMesh(core_axis_name="c", subcore_axis_name="s"),
    scratch_types=[
        pltpu.VMEM((E_LO + CHUNK,), jnp.int32),
        pltpu.VMEM((E_LO + CHUNK,), jnp.int32),
        pltpu.VMEM((E_LO + CHUNK,), jnp.int32),
        pltpu.VMEM((CHUNK,), jnp.int32),
        pltpu.VMEM((CHUNK,), jnp.int32),
        pltpu.VMEM((CHUNK,), jnp.int32),
        pltpu.VMEM((CHUNK,), jnp.int32),
        pltpu.VMEM((CHUNK, D), jnp.float32),
        pltpu.VMEM((CHUNK, D), jnp.float32),
        pltpu.VMEM_SHARED((N, D), jnp.float32),
        pltpu.SemaphoreType.DMA,
        pltpu.SemaphoreType.DMA,
    ],
)


@jax.jit
def kernel(edge_index, edge_type, embeddings, W, b):
    zeros = jnp.zeros((ROWS_PER_TILE, D), jnp.float32)
    b4 = b.reshape(L, C, 1, D)

    y = _transform(embeddings, W, b4)
    ha, hb = _sc_scatter(y.reshape(C * N, D), edge_index, edge_type, zeros)

    y = _transform_relu(ha, hb, W, b4)
    ha, hb = _sc_scatter(y.reshape(C * N, D), edge_index, edge_type, zeros)

    return _add(ha, hb)
